# baseline (speedup 1.0000x reference)
import jax
import jax.numpy as jnp
from jax import lax
from jax.experimental import pallas as pl
from jax.experimental.pallas import tpu as pltpu
from jax.experimental.pallas import tpu_sc as plsc

_B = 16384
_CF = 16
_SUB = _CF * _CF
_NV = 1000000
_NC, _NS = 2, 16
_NW = _NC * _NS
_BPW = _B // _NW
_CHUNK = 16
_NCHUNK = _BPW // _CHUNK
_LAST_BLK = _NV // 128 - 1
_TAIL_START = _NV - 128
_TAIL_CUT = (_LAST_BLK + 1) * 128
_TAIL_OFF = _CHUNK * 128


def _gather_body(u_idx, i_idx, u_tab, i_tab, u_tail, i_tail, u_out, i_out,
                 uidx_v, iidx_v, ublk, iblk, ustg, istg,
                 usem, isem, uosem, iosem):
    wid = lax.axis_index("s") * _NC + lax.axis_index("c")
    base = pl.multiple_of(wid * _BPW, 128)
    pltpu.sync_copy(u_idx.at[pl.ds(base, _BPW)], uidx_v)
    pltpu.sync_copy(i_idx.at[pl.ds(base, _BPW)], iidx_v)
    pltpu.sync_copy(u_tail, ublk.at[:, pl.ds(_TAIL_OFF, 128)])
    pltpu.sync_copy(i_tail, iblk.at[:, pl.ds(_TAIL_OFF, 128)])

    lastb = jnp.full((_CHUNK,), _LAST_BLK, jnp.int32)

    def fire(tab, idx_v, blk, sem, g):
        iv = idx_v[pl.ds(g * _CHUNK, _CHUNK)]
        bv = jnp.minimum(lax.shift_right_logical(iv, 7), lastb)
        for k in range(_CHUNK):
            pltpu.async_copy(
                tab.at[:, pl.ds(pl.multiple_of(bv[k] * 128, 128), 128)],
                blk.at[:, pl.ds(k * 128, 128)], sem)

    def drain_blocks(tab, blk, sem):
        pltpu.make_async_copy(tab.at[:, pl.ds(0, _CHUNK * 128)],
                              blk.at[:, pl.ds(0, _CHUNK * 128)], sem).wait()

    def stage(idx_v, blk, stg, g):
        iv = idx_v[pl.ds(g * _CHUNK, _CHUNK)]
        bv = jnp.minimum(lax.shift_right_logical(iv, 7), lastb)
        slotv = lax.iota(jnp.int32, _CHUNK) * 128 + (iv - bv * 128)
        tailv = _TAIL_OFF + (iv - _TAIL_START)
        absv = jnp.where(iv >= _TAIL_CUT, tailv, slotv)
        startv = lax.shift_left(lax.shift_right_logical(absv, 4), 4)
        for k in range(_CHUNK):
            start = pl.multiple_of(startv[k], 16)
            for d in range(_CF):
                stg[k, pl.ds(d * _CF, _CF)] = blk[d, pl.ds(start, 16)]

    def drain_out(stg, out, osem):
        pltpu.make_async_copy(stg, out.at[pl.ds(0, _CHUNK), :], osem).wait()

    fire(u_tab, uidx_v, ublk, usem, 0)
    fire(i_tab, iidx_v, iblk, isem, 0)

    def body(g, carry):
        row0 = pl.multiple_of(base + g * _CHUNK, 16)
        drain_blocks(u_tab, ublk, usem)

        @pl.when(g > 0)
        def _():
            drain_out(ustg, u_out, uosem)

        stage(uidx_v, ublk, ustg, g)
        pltpu.async_copy(ustg, u_out.at[pl.ds(row0, _CHUNK), :], uosem)

        @pl.when(g < _NCHUNK - 1)
        def _():
            fire(u_tab, uidx_v, ublk, usem, g + 1)

        drain_blocks(i_tab, iblk, isem)

        @pl.when(g > 0)
        def _():
            drain_out(istg, i_out, iosem)

        stage(iidx_v, iblk, istg, g)
        pltpu.async_copy(istg, i_out.at[pl.ds(row0, _CHUNK), :], iosem)

        @pl.when(g < _NCHUNK - 1)
        def _():
            fire(i_tab, iidx_v, iblk, isem, g + 1)

        return carry

    lax.fori_loop(0, _NCHUNK, body, 0)
    drain_out(ustg, u_out, uosem)
    drain_out(istg, i_out, iosem)


def _make_gather():
    return pl.kernel(
        _gather_body,
        out_type=(jax.ShapeDtypeStruct((_B, _SUB), jnp.float32),
                  jax.ShapeDtypeStruct((_B, _SUB), jnp.float32)),
        mesh=plsc.VectorSubcoreMesh(core_axis_name="c", subcore_axis_name="s",
                                    num_cores=_NC, num_subcores=_NS),
        scratch_types=[
            pltpu.VMEM((_BPW,), jnp.int32),
            pltpu.VMEM((_BPW,), jnp.int32),
            pltpu.VMEM((_CF, (_CHUNK + 1) * 128), jnp.float32),
            pltpu.VMEM((_CF, (_CHUNK + 1) * 128), jnp.float32),
            pltpu.VMEM((_CHUNK, _SUB), jnp.float32),
            pltpu.VMEM((_CHUNK, _SUB), jnp.float32),
            pltpu.SemaphoreType.DMA,
            pltpu.SemaphoreType.DMA,
            pltpu.SemaphoreType.DMA,
            pltpu.SemaphoreType.DMA,
        ],
    )


_BLK = 2048


def _dot(a, b):
    return jnp.dot(a, b, preferred_element_type=jnp.float32,
                   precision=lax.Precision.HIGHEST)


def _mlp_body(feat, subu, subi, ru, ri, cw0, cb0, cw1, cb1, cw2, cb2,
              swu, swi, w0c, b0, w1, b1, w2, b2, fwt, fb, out):
    f32 = jnp.float32
    h = jnp.maximum(_dot(feat[...], cw0[...]) + cb0[...], 0.0)
    h = jnp.maximum(_dot(h, cw1[...]) + cb1[...], 0.0)
    c = _dot(h, cw2[...]) + cb2[...]
    sel = lax.broadcasted_iota(jnp.int32, (_BLK, _SUB), 1) & 15
    ohu = (sel == ru[...]).astype(f32)
    ohi = (sel == ri[...]).astype(f32)
    m = (_dot(subu[...] * ohu, swu[...])
         + _dot(subi[...] * ohi, swi[...])
         + _dot(c, w0c[...]))
    m = jnp.maximum(m + b0[...], 0.0)
    m = jnp.maximum(_dot(m, w1[...]) + b1[...], 0.0)
    m = jnp.maximum(_dot(m, w2[...]) + b2[...], 0.0)
    out[...] = jnp.sum(m * fwt[...], axis=1) + fb[...]


def _full(shape):
    if len(shape) == 1:
        return pl.BlockSpec(shape, lambda i: (0,))
    return pl.BlockSpec(shape, lambda i: (0, 0))


def _mlp(feat, subu, subi, ru, ri, cw0, cb0, cw1, cb1, cw2, cb2,
         swu, swi, w0c, b0, w1, b1, w2, b2, fwt, fb):
    grid = (_B // _BLK,)
    return pl.pallas_call(
        _mlp_body,
        grid=grid,
        in_specs=[
            pl.BlockSpec((_BLK, 128), lambda i: (i, 0)),
            pl.BlockSpec((_BLK, _SUB), lambda i: (i, 0)),
            pl.BlockSpec((_BLK, _SUB), lambda i: (i, 0)),
            pl.BlockSpec((_BLK, 1), lambda i: (i, 0)),
            pl.BlockSpec((_BLK, 1), lambda i: (i, 0)),
            _full((128, 32)), _full((1, 32)),
            _full((32, 16)), _full((1, 16)),
            _full((16, _CF)), _full((1, _CF)),
            _full((_SUB, 32)), _full((_SUB, 32)), _full((_CF, 32)), _full((1, 32)),
            _full((32, 16)), _full((1, 16)),
            _full((16, 8)), _full((1, 8)),
            _full((1, 8)), _full((1,)),
        ],
        out_specs=pl.BlockSpec((_BLK,), lambda i: (i,)),
        out_shape=jax.ShapeDtypeStruct((_B,), jnp.float32),
    )(feat, subu, subi, ru, ri, cw0, cb0, cw1, cb1, cw2, cb2,
      swu, swi, w0c, b0, w1, b1, w2, b2, fwt, fb)


def kernel(user_indices, item_indices, item_features, user_table, item_table,
           ce_w0, ce_b0, ce_w1, ce_b1, ce_w2, ce_b2,
           mlp_w0, mlp_b0, mlp_w1, mlp_b1, mlp_w2, mlp_b2,
           fin_w, fin_b):
    u_tail = user_table[_TAIL_START:].T
    i_tail = item_table[_TAIL_START:].T
    subu, subi = _make_gather()(
        user_indices, item_indices, user_table.T, item_table.T, u_tail, i_tail)
    ru = (user_indices & 15).astype(jnp.int32).reshape(_B, 1)
    ri = (item_indices & 15).astype(jnp.int32).reshape(_B, 1)
    swu = jnp.repeat(mlp_w0[:_CF], _CF, axis=0)
    swi = jnp.repeat(mlp_w0[_CF:2 * _CF], _CF, axis=0)
    return _mlp(
        item_features, subu, subi, ru, ri,
        ce_w0, ce_b0.reshape(1, -1), ce_w1, ce_b1.reshape(1, -1),
        ce_w2, ce_b2.reshape(1, -1),
        swu, swi, mlp_w0[2 * _CF:],
        mlp_b0.reshape(1, -1), mlp_w1, mlp_b1.reshape(1, -1),
        mlp_w2, mlp_b2.reshape(1, -1),
        fin_w.reshape(1, -1), fin_b,
    )


# speedup vs baseline: 1.3409x; 1.3409x over previous
import jax
import jax.numpy as jnp
from jax import lax
from jax.experimental import pallas as pl
from jax.experimental.pallas import tpu as pltpu
from jax.experimental.pallas import tpu_sc as plsc

_B = 16384
_CF = 16
_SUB = _CF * _CF
_NV = 1000000
_NC, _NS = 2, 16
_NW = _NC * _NS
_BPW = _B // _NW
_CHUNK = 16
_NCHUNK = _BPW // _CHUNK
_LAST_BLK = _NV // 128 - 1
_TAIL_START = _NV - 128
_TAIL_CUT = (_LAST_BLK + 1) * 128
_TAIL_OFF = _CHUNK * 128


def _gather_body(u_idx, i_idx, u_tab, i_tab, u_tail, i_tail, u_out, i_out,
                 uidx_v, iidx_v, ublk, iblk, ustg, istg,
                 usem, isem, uosem, iosem):
    wid = lax.axis_index("s") * _NC + lax.axis_index("c")
    base = pl.multiple_of(wid * _BPW, 128)
    pltpu.sync_copy(u_idx.at[pl.ds(base, _BPW)], uidx_v)
    pltpu.sync_copy(i_idx.at[pl.ds(base, _BPW)], iidx_v)
    pltpu.sync_copy(u_tail, ublk.at[:, pl.ds(_TAIL_OFF, 128)])
    pltpu.sync_copy(i_tail, iblk.at[:, pl.ds(_TAIL_OFF, 128)])

    lastb = jnp.full((_CHUNK,), _LAST_BLK, jnp.int32)

    def fire(tab, idx_v, blk, sem, g):
        iv = idx_v[pl.ds(g * _CHUNK, _CHUNK)]
        bv = jnp.minimum(lax.shift_right_logical(iv, 7), lastb)
        for k in range(_CHUNK):
            pltpu.async_copy(
                tab.at[:, pl.ds(pl.multiple_of(bv[k] * 128, 128), 128)],
                blk.at[:, pl.ds(k * 128, 128)], sem)

    def drain_blocks(tab, blk, sem):
        pltpu.make_async_copy(tab.at[:, pl.ds(0, _CHUNK * 128)],
                              blk.at[:, pl.ds(0, _CHUNK * 128)], sem).wait()

    def stage(idx_v, blk, stg, g):
        iv = idx_v[pl.ds(g * _CHUNK, _CHUNK)]
        bv = jnp.minimum(lax.shift_right_logical(iv, 7), lastb)
        slotv = lax.iota(jnp.int32, _CHUNK) * 128 + (iv - bv * 128)
        tailv = _TAIL_OFF + (iv - _TAIL_START)
        absv = jnp.where(iv >= _TAIL_CUT, tailv, slotv)
        startv = lax.shift_left(lax.shift_right_logical(absv, 4), 4)
        for k in range(_CHUNK):
            start = pl.multiple_of(startv[k], 16)
            for d in range(_CF):
                stg[k, pl.ds(d * _CF, _CF)] = blk[d, pl.ds(start, 16)]

    def drain_out(stg, out, osem):
        pltpu.make_async_copy(stg, out.at[pl.ds(0, _CHUNK), :], osem).wait()

    fire(u_tab, uidx_v, ublk, usem, 0)
    fire(i_tab, iidx_v, iblk, isem, 0)

    def body(g, carry):
        row0 = pl.multiple_of(base + g * _CHUNK, 16)
        drain_blocks(u_tab, ublk, usem)

        @pl.when(g > 0)
        def _():
            drain_out(ustg, u_out, uosem)

        stage(uidx_v, ublk, ustg, g)
        pltpu.async_copy(ustg, u_out.at[pl.ds(row0, _CHUNK), :], uosem)

        @pl.when(g < _NCHUNK - 1)
        def _():
            fire(u_tab, uidx_v, ublk, usem, g + 1)

        drain_blocks(i_tab, iblk, isem)

        @pl.when(g > 0)
        def _():
            drain_out(istg, i_out, iosem)

        stage(iidx_v, iblk, istg, g)
        pltpu.async_copy(istg, i_out.at[pl.ds(row0, _CHUNK), :], iosem)

        @pl.when(g < _NCHUNK - 1)
        def _():
            fire(i_tab, iidx_v, iblk, isem, g + 1)

        return carry

    lax.fori_loop(0, _NCHUNK, body, 0)
    drain_out(ustg, u_out, uosem)
    drain_out(istg, i_out, iosem)


def _make_gather():
    return pl.kernel(
        _gather_body,
        out_type=(jax.ShapeDtypeStruct((_B, _SUB), jnp.float32),
                  jax.ShapeDtypeStruct((_B, _SUB), jnp.float32)),
        mesh=plsc.VectorSubcoreMesh(core_axis_name="c", subcore_axis_name="s",
                                    num_cores=_NC, num_subcores=_NS),
        scratch_types=[
            pltpu.VMEM((_BPW,), jnp.int32),
            pltpu.VMEM((_BPW,), jnp.int32),
            pltpu.VMEM((_CF, (_CHUNK + 1) * 128), jnp.float32),
            pltpu.VMEM((_CF, (_CHUNK + 1) * 128), jnp.float32),
            pltpu.VMEM((_CHUNK, _SUB), jnp.float32),
            pltpu.VMEM((_CHUNK, _SUB), jnp.float32),
            pltpu.SemaphoreType.DMA,
            pltpu.SemaphoreType.DMA,
            pltpu.SemaphoreType.DMA,
            pltpu.SemaphoreType.DMA,
        ],
    )


_BLK = 2048


def _dot(a, b):
    return jnp.dot(a, b, preferred_element_type=jnp.float32)


def _mlp_body(feat, subu, subi, ru, ri, cw0, cb0, cw1, cb1, cw2, cb2,
              selm, w0, b0, w1, b1, w2, b2, fw, fb, out):
    f32 = jnp.float32
    h = jnp.maximum(_dot(feat[...], cw0[...]) + cb0[...], 0.0)
    h = jnp.maximum(_dot(h, cw1[...]) + cb1[...], 0.0)
    c = _dot(h, cw2[...]) + cb2[...]
    sel = lax.broadcasted_iota(jnp.int32, (_BLK, _SUB), 1) & 15
    ohu = (sel == ru[...]).astype(f32)
    ohi = (sel == ri[...]).astype(f32)
    uemb = jnp.dot(subu[...] * ohu, selm[...], preferred_element_type=f32,
                   precision=lax.Precision.HIGHEST)
    iemb = jnp.dot(subi[...] * ohi, selm[...], preferred_element_type=f32,
                   precision=lax.Precision.HIGHEST)
    combined = jnp.concatenate([uemb, iemb, c], axis=1)
    m = jnp.maximum(_dot(combined, w0[...]) + b0[...], 0.0)
    m = jnp.maximum(_dot(m, w1[...]) + b1[...], 0.0)
    m = jnp.maximum(_dot(m, w2[...]) + b2[...], 0.0)
    out[...] = (_dot(m, fw[...]) + fb[...])[:, 0]


def _full(shape):
    if len(shape) == 1:
        return pl.BlockSpec(shape, lambda i: (0,))
    return pl.BlockSpec(shape, lambda i: (0, 0))


def _mlp(feat, subu, subi, ru, ri, cw0, cb0, cw1, cb1, cw2, cb2,
         selm, w0, b0, w1, b1, w2, b2, fw, fb):
    grid = (_B // _BLK,)
    return pl.pallas_call(
        _mlp_body,
        grid=grid,
        in_specs=[
            pl.BlockSpec((_BLK, 128), lambda i: (i, 0)),
            pl.BlockSpec((_BLK, _SUB), lambda i: (i, 0)),
            pl.BlockSpec((_BLK, _SUB), lambda i: (i, 0)),
            pl.BlockSpec((_BLK, 1), lambda i: (i, 0)),
            pl.BlockSpec((_BLK, 1), lambda i: (i, 0)),
            _full((128, 32)), _full((1, 32)),
            _full((32, 16)), _full((1, 16)),
            _full((16, _CF)), _full((1, _CF)),
            _full((_SUB, _CF)),
            _full((3 * _CF, 32)), _full((1, 32)),
            _full((32, 16)), _full((1, 16)),
            _full((16, 8)), _full((1, 8)),
            _full((8, 1)), _full((1,)),
        ],
        out_specs=pl.BlockSpec((_BLK,), lambda i: (i,)),
        out_shape=jax.ShapeDtypeStruct((_B,), jnp.float32),
    )(feat, subu, subi, ru, ri, cw0, cb0, cw1, cb1, cw2, cb2,
      selm, w0, b0, w1, b1, w2, b2, fw, fb)


def kernel(user_indices, item_indices, item_features, user_table, item_table,
           ce_w0, ce_b0, ce_w1, ce_b1, ce_w2, ce_b2,
           mlp_w0, mlp_b0, mlp_w1, mlp_b1, mlp_w2, mlp_b2,
           fin_w, fin_b):
    u_tail = user_table[_TAIL_START:].T
    i_tail = item_table[_TAIL_START:].T
    subu, subi = _make_gather()(
        user_indices, item_indices, user_table.T, item_table.T, u_tail, i_tail)
    ru = (user_indices & 15).astype(jnp.int32).reshape(_B, 1)
    ri = (item_indices & 15).astype(jnp.int32).reshape(_B, 1)
    selm = jnp.repeat(jnp.eye(_CF, dtype=jnp.float32), _CF, axis=0)
    return _mlp(
        item_features, subu, subi, ru, ri,
        ce_w0, ce_b0.reshape(1, -1), ce_w1, ce_b1.reshape(1, -1),
        ce_w2, ce_b2.reshape(1, -1),
        selm, mlp_w0,
        mlp_b0.reshape(1, -1), mlp_w1, mlp_b1.reshape(1, -1),
        mlp_w2, mlp_b2.reshape(1, -1),
        fin_w, fin_b,
    )
